# baseline (speedup 1.0000x reference)
import functools

import jax
import jax.numpy as jnp
from jax import lax
from jax.experimental import pallas as pl
from jax.experimental.pallas import tpu as pltpu
from jax.experimental.pallas import tpu_sc as plsc

N_FACTORS = 64
BATCH = 16384
_LANES = 16
_CHUNK = 128


def _make_sc_call():
    info = plsc.get_sparse_core_info()
    nc, ns = info.num_cores, info.num_subcores
    nw = nc * ns
    rows = BATCH // nw
    groups = rows // _LANES
    n_chunks = rows // _CHUNK

    mesh = plsc.VectorSubcoreMesh(core_axis_name="c", subcore_axis_name="s")

    @functools.partial(
        pl.kernel,
        mesh=mesh,
        compiler_params=pltpu.CompilerParams(needs_layout_passes=False,
                                             use_tc_tiling_on_sc=False),
        out_type=jax.ShapeDtypeStruct((BATCH,), jnp.float32),
        scratch_types=[
            pltpu.VMEM((rows * 2,), jnp.int32),
            pltpu.VMEM((rows,), jnp.int32),
            pltpu.VMEM((rows,), jnp.int32),
            pltpu.VMEM((rows, N_FACTORS), jnp.float32),
            pltpu.VMEM((rows, N_FACTORS), jnp.float32),
            pltpu.VMEM((rows,), jnp.float32),
            pltpu.SemaphoreType.DMA,
        ],
    )
    def sc_call(cats_hbm, u_hbm, b_hbm, out_hbm,
                cats_v, uidx_v, bidx_v, urows_v, brows_v, out_v, sem):
        wid = lax.axis_index("s") * nc + lax.axis_index("c")
        base = wid * rows

        pltpu.sync_copy(cats_hbm.at[pl.ds(base * 2, rows * 2)], cats_v)

        iota = lax.iota(jnp.int32, _LANES)

        def deint_body(c, carry):
            f16 = (c * _LANES + iota) * 2
            uidx_v[pl.ds(c * _LANES, _LANES)] = plsc.load_gather(
                cats_v, [f16])
            bidx_v[pl.ds(c * _LANES, _LANES)] = plsc.load_gather(
                cats_v, [f16 + 1])
            return carry

        lax.fori_loop(0, groups, deint_body, 0)

        copies = []
        for c in range(n_chunks):
            sl = pl.ds(c * _CHUNK, _CHUNK)
            copies.append(pltpu.async_copy(
                u_hbm.at[uidx_v.at[sl]], urows_v.at[sl, :], sem))
            copies.append(pltpu.async_copy(
                b_hbm.at[bidx_v.at[sl]], brows_v.at[sl, :], sem))
        for cp in copies:
            cp.wait()

        def dot_body(g, carry):
            r16 = g * _LANES + iota
            acc = jnp.zeros((_LANES,), jnp.float32)
            for d in range(N_FACTORS):
                dvec = jnp.full((_LANES,), d, jnp.int32)
                uv = plsc.load_gather(urows_v, [r16, dvec])
                bv = plsc.load_gather(brows_v, [r16, dvec])
                acc = acc + uv * bv
            out_v[pl.ds(g * _LANES, _LANES)] = acc
            return carry

        lax.fori_loop(0, groups, dot_body, 0)

        pltpu.sync_copy(out_v, out_hbm.at[pl.ds(base, rows)])

    return sc_call


def kernel(cats, conts, U, B):
    del conts
    return _make_sc_call()(cats.reshape(-1), U, B)


# speedup vs baseline: 1.2193x; 1.2193x over previous
import functools

import jax
import jax.numpy as jnp
from jax import lax
from jax.experimental import pallas as pl
from jax.experimental.pallas import tpu as pltpu
from jax.experimental.pallas import tpu_sc as plsc

N_FACTORS = 64
BATCH = 16384
_LANES = 16
_CHUNK = 128
_PITCH = 17


def _make_sc_call():
    info = plsc.get_sparse_core_info()
    nc, ns = info.num_cores, info.num_subcores
    nw = nc * ns
    rows = BATCH // nw
    groups = rows // _LANES
    n_chunks = rows // _CHUNK

    mesh = plsc.VectorSubcoreMesh(core_axis_name="c", subcore_axis_name="s")

    @functools.partial(
        pl.kernel,
        mesh=mesh,
        compiler_params=pltpu.CompilerParams(needs_layout_passes=False,
                                             use_tc_tiling_on_sc=False),
        out_type=jax.ShapeDtypeStruct((BATCH,), jnp.float32),
        scratch_types=[
            pltpu.VMEM((rows,), jnp.int32),
            pltpu.VMEM((rows,), jnp.int32),
            pltpu.VMEM((rows, N_FACTORS), jnp.float32),
            pltpu.VMEM((rows, N_FACTORS), jnp.float32),
            pltpu.VMEM((_LANES * _PITCH,), jnp.float32),
            pltpu.VMEM((rows,), jnp.float32),
            pltpu.SemaphoreType.DMA,
        ],
    )
    def sc_call(cats_hbm, u_hbm, b_hbm, out_hbm,
                uidx_v, bidx_v, urows_v, brows_v, stage_v, out_v, sem):
        wid = lax.axis_index("s") * nc + lax.axis_index("c")
        base = wid * rows

        pltpu.sync_copy(cats_hbm.at[pl.ds(base, rows)], uidx_v)
        pltpu.sync_copy(cats_hbm.at[pl.ds(BATCH + base, rows)], bidx_v)

        copies = []
        for c in range(n_chunks):
            sl = pl.ds(c * _CHUNK, _CHUNK)
            copies.append(pltpu.async_copy(
                u_hbm.at[uidx_v.at[sl]], urows_v.at[sl, :], sem))
            copies.append(pltpu.async_copy(
                b_hbm.at[bidx_v.at[sl]], brows_v.at[sl, :], sem))
        for cp in copies:
            cp.wait()

        iota17 = lax.iota(jnp.int32, _LANES) * _PITCH

        def dot_body(g, carry):
            r0 = g * _LANES
            for r in range(_LANES):
                row = r0 + r
                partial = (urows_v[row, pl.ds(0, _LANES)]
                           * brows_v[row, pl.ds(0, _LANES)])
                for c in range(1, N_FACTORS // _LANES):
                    partial = partial + (
                        urows_v[row, pl.ds(c * _LANES, _LANES)]
                        * brows_v[row, pl.ds(c * _LANES, _LANES)])
                stage_v[pl.ds(r * _PITCH, _LANES)] = partial
            acc = plsc.load_gather(stage_v, [iota17])
            for k in range(1, _LANES):
                acc = acc + plsc.load_gather(stage_v, [iota17 + k])
            out_v[pl.ds(r0, _LANES)] = acc
            return carry

        lax.fori_loop(0, groups, dot_body, 0)

        pltpu.sync_copy(out_v, out_hbm.at[pl.ds(base, rows)])

    return sc_call


def kernel(cats, conts, U, B):
    del conts
    return _make_sc_call()(cats.T.reshape(-1), U, B)
